# baseline (speedup 1.0000x reference)
import functools

import jax
import jax.numpy as jnp
from jax import lax
from jax.experimental import pallas as pl
from jax.experimental.pallas import tpu as pltpu
from jax.experimental.pallas import tpu_sc as plsc

_D = 32
_NC = 2
_NS = 16
_NW = _NC * _NS
_S = 2
_NBUF = 4
_HP = 56
_DP = 128


_GC = 25
_TCH = 158


@functools.lru_cache(maxsize=None)
def _build_compact(vocab: int):
    rows_per_chunk = _GC * 8
    n_chunk_total = vocab // rows_per_chunk
    assert vocab % rows_per_chunk == 0 and _TCH * _NW >= n_chunk_total
    mesh = plsc.VectorSubcoreMesh(core_axis_name="c", subcore_axis_name="s")

    @functools.partial(
        pl.kernel,
        mesh=mesh,
        out_type=jax.ShapeDtypeStruct((vocab * _D,), jnp.float32),
        scratch_types=[
            pltpu.VMEM((2, rows_per_chunk, _D), jnp.float32),
            pltpu.VMEM((2, rows_per_chunk * _D), jnp.float32),
            [pltpu.SemaphoreType.DMA] * 2,
            [pltpu.SemaphoreType.DMA] * 2,
        ],
    )
    def compact(tab_hbm, flat_hbm, vbuf, cbuf, isem, osem):
        wid = lax.axis_index("s") * _NC + lax.axis_index("c")

        def chunk_r0(j):
            return jnp.minimum((wid * _TCH + j) * rows_per_chunk,
                               vocab - rows_per_chunk)

        def i_copy(j, b):
            return pltpu.make_async_copy(
                tab_hbm.at[pl.ds(chunk_r0(j), rows_per_chunk), :], vbuf.at[b],
                isem[b])

        def o_copy(j, b):
            return pltpu.make_async_copy(
                cbuf.at[b],
                flat_hbm.at[pl.ds(chunk_r0(j) * _D, rows_per_chunk * _D)],
                osem[b])

        def extract(b):
            for r in range(rows_per_chunk):
                for h in range(0, _D, 16):
                    cbuf[b, pl.ds(r * _D + h, 16)] = vbuf[b, r, pl.ds(h, 16)]

        i_copy(0, 0).start()
        i_copy(1, 1).start()

        def pair(j, launch_next):
            for b in range(2):
                jj = 2 * j + b
                i_copy(jj, b).wait()
                extract(b)
                o_copy(jj, b).start()
                o_copy(jj, b).wait()
                if launch_next:
                    i_copy(jj + 2, b).start()

        def body(j, carry):
            pair(j, True)
            return carry

        lax.fori_loop(0, _TCH // 2 - 1, body, 0)
        pair(_TCH // 2 - 1, False)

    return compact


@functools.lru_cache(maxsize=None)
def _build(batch: int, hist: int):
    s_per_w = batch // _NW
    n_chunk = s_per_w // _S
    n_grp = n_chunk // _NBUF
    assert n_chunk % _NBUF == 0 and n_grp >= 3
    mesh = plsc.VectorSubcoreMesh(core_axis_name="c", subcore_axis_name="s")

    @functools.partial(
        pl.kernel,
        mesh=mesh,
        out_type=jax.ShapeDtypeStruct((batch, _HP, _DP), jnp.float32),
        scratch_types=[
            pltpu.VMEM((s_per_w, hist), jnp.int32),
            pltpu.VMEM((_NBUF, _S, hist, _D), jnp.float32),
            pltpu.VMEM((_NBUF, _S, _HP, _DP), jnp.float32),
            [pltpu.SemaphoreType.DMA] * _NBUF,
            [pltpu.SemaphoreType.DMA] * _NBUF,
        ],
        compiler_params=pltpu.CompilerParams(use_tc_tiling_on_sc=False),
    )
    def gather(x_hbm, table_hbm, out_hbm, idx_v, gbuf, pbuf, gsem, wsem):
        wid = lax.axis_index("s") * _NC + lax.axis_index("c")
        base = wid * s_per_w

        def g_copy_one(i, b, k):
            return pltpu.make_async_copy(
                table_hbm.at[idx_v.at[i * _S + k, :]],
                gbuf.at[b].at[k], gsem[b])

        def g_start(i, b):
            for k in range(_S):
                g_copy_one(i, b, k).start()

        def g_wait(i, b):
            for k in range(_S):
                g_copy_one(i, b, k).wait()

        def repack(b):
            for k in range(_S):
                for j in range(hist):
                    for h in range(0, _D, 16):
                        pbuf[b, k, j, pl.ds(h, 16)] = (
                            gbuf[b, k, j, pl.ds(h, 16)])

        def w_copy(i, b):
            return pltpu.make_async_copy(
                pbuf.at[b], out_hbm.at[pl.ds(base + i * _S, _S), :, :],
                wsem[b])

        pltpu.sync_copy(x_hbm.at[pl.ds(base, s_per_w), :], idx_v)

        for b in range(_NBUF):
            g_start(b, b)

        def slots(i0, first, launch_next):
            for b in range(_NBUF):
                i = i0 + b
                g_wait(i, b)
                if not first:
                    w_copy(i - _NBUF, b).wait()
                repack(b)
                w_copy(i, b).start()
                if launch_next:
                    g_start(i + _NBUF, b)

        slots(0, True, True)

        def body(j, carry):
            slots(j * _NBUF, False, True)
            return carry

        lax.fori_loop(1, n_grp - 1, body, 0)
        slots((n_grp - 1) * _NBUF, False, False)
        for b in range(_NBUF):
            w_copy((n_grp - 1) * _NBUF + b, b).wait()

    return gather


def kernel(X, table):
    B, H = X.shape
    V = table.shape[0]
    tab_flat = _build_compact(V)(table)
    out_pad = _build(B, H)(X.astype(jnp.int32), tab_flat.reshape(V, _D))
    return out_pad[:, :H, :_D]


# speedup vs baseline: 1.1566x; 1.1566x over previous
import functools

import jax
import jax.numpy as jnp
from jax import lax
from jax.experimental import pallas as pl
from jax.experimental.pallas import tpu as pltpu
from jax.experimental.pallas import tpu_sc as plsc

_D = 32
_NC = 2
_NS = 16
_NW = _NC * _NS
_S = 2
_NBUF = 4
_HP = 56
_DP = 128


_GC = 25
_TCH = 158


@functools.lru_cache(maxsize=None)
def _build_compact(vocab: int):
    rows_per_chunk = _GC * 8
    n_chunk_total = vocab // rows_per_chunk
    assert vocab % rows_per_chunk == 0 and _TCH * _NW >= n_chunk_total
    mesh = plsc.VectorSubcoreMesh(core_axis_name="c", subcore_axis_name="s")

    @functools.partial(
        pl.kernel,
        mesh=mesh,
        out_type=jax.ShapeDtypeStruct((vocab * _D,), jnp.float32),
        scratch_types=[
            pltpu.VMEM((2, _GC, 8, _D), jnp.float32),
            pltpu.VMEM((2, rows_per_chunk * _D), jnp.float32),
            [pltpu.SemaphoreType.DMA] * 2,
            [pltpu.SemaphoreType.DMA] * 2,
        ],
    )
    def compact(tab_hbm, flat_hbm, vbuf, cbuf, isem, osem):
        wid = lax.axis_index("s") * _NC + lax.axis_index("c")

        def chunk_g0(j):
            return jnp.minimum((wid * _TCH + j) * _GC, vocab // 8 - _GC)

        def i_copy(j, b):
            return pltpu.make_async_copy(
                tab_hbm.at[pl.ds(chunk_g0(j), _GC), :, :], vbuf.at[b],
                isem[b])

        def o_copy(j, b):
            return pltpu.make_async_copy(
                cbuf.at[b],
                flat_hbm.at[pl.ds(chunk_g0(j) * 8 * _D, rows_per_chunk * _D)],
                osem[b])

        def extract(b):
            for g in range(_GC):
                for r in range(8):
                    n = (g * 8 + r) * _D
                    for h in range(0, _D, 16):
                        cbuf[b, pl.ds(n + h, 16)] = vbuf[b, g, r, pl.ds(h, 16)]

        i_copy(0, 0).start()
        i_copy(1, 1).start()

        def pair(j, launch_next):
            for b in range(2):
                jj = 2 * j + b
                i_copy(jj, b).wait()
                extract(b)
                o_copy(jj, b).start()
                o_copy(jj, b).wait()
                if launch_next:
                    i_copy(jj + 2, b).start()

        def body(j, carry):
            pair(j, True)
            return carry

        lax.fori_loop(0, _TCH // 2 - 1, body, 0)
        pair(_TCH // 2 - 1, False)

    return compact


@functools.lru_cache(maxsize=None)
def _build(batch: int, hist: int):
    s_per_w = batch // _NW
    n_chunk = s_per_w // _S
    n_grp = n_chunk // _NBUF
    assert n_chunk % _NBUF == 0 and n_grp >= 3
    mesh = plsc.VectorSubcoreMesh(core_axis_name="c", subcore_axis_name="s")

    @functools.partial(
        pl.kernel,
        mesh=mesh,
        out_type=jax.ShapeDtypeStruct((batch, _HP, _DP), jnp.float32),
        scratch_types=[
            pltpu.VMEM((s_per_w, hist), jnp.int32),
            pltpu.VMEM((_NBUF, _S, hist, _D), jnp.float32),
            pltpu.VMEM((_NBUF, _S, _HP, _DP), jnp.float32),
            [pltpu.SemaphoreType.DMA] * _NBUF,
            [pltpu.SemaphoreType.DMA] * _NBUF,
        ],
        compiler_params=pltpu.CompilerParams(use_tc_tiling_on_sc=False),
    )
    def gather(x_hbm, table_hbm, out_hbm, idx_v, gbuf, pbuf, gsem, wsem):
        wid = lax.axis_index("s") * _NC + lax.axis_index("c")
        base = wid * s_per_w

        def g_copy_one(i, b, k):
            return pltpu.make_async_copy(
                table_hbm.at[idx_v.at[i * _S + k, :]],
                gbuf.at[b].at[k], gsem[b])

        def g_start(i, b):
            for k in range(_S):
                g_copy_one(i, b, k).start()

        def g_wait(i, b):
            for k in range(_S):
                g_copy_one(i, b, k).wait()

        def repack(b):
            for k in range(_S):
                for j in range(hist):
                    for h in range(0, _D, 16):
                        pbuf[b, k, j, pl.ds(h, 16)] = (
                            gbuf[b, k, j, pl.ds(h, 16)])

        def w_copy(i, b):
            return pltpu.make_async_copy(
                pbuf.at[b], out_hbm.at[pl.ds(base + i * _S, _S), :, :],
                wsem[b])

        pltpu.sync_copy(x_hbm.at[pl.ds(base, s_per_w), :], idx_v)

        for b in range(_NBUF):
            g_start(b, b)

        def slots(i0, first, launch_next):
            for b in range(_NBUF):
                i = i0 + b
                g_wait(i, b)
                if not first:
                    w_copy(i - _NBUF, b).wait()
                repack(b)
                w_copy(i, b).start()
                if launch_next:
                    g_start(i + _NBUF, b)

        slots(0, True, True)

        def body(j, carry):
            slots(j * _NBUF, False, True)
            return carry

        lax.fori_loop(1, n_grp - 1, body, 0)
        slots((n_grp - 1) * _NBUF, False, False)
        for b in range(_NBUF):
            w_copy((n_grp - 1) * _NBUF + b, b).wait()

    return gather


def kernel(X, table):
    B, H = X.shape
    V = table.shape[0]
    tab_flat = _build_compact(V)(table.reshape(V // 8, 8, _D))
    out_pad = _build(B, H)(X.astype(jnp.int32), tab_flat.reshape(V, _D))
    return out_pad[:, :H, :_D]


# speedup vs baseline: 1.1710x; 1.0124x over previous
import functools

import jax
import jax.numpy as jnp
from jax import lax
from jax.experimental import pallas as pl
from jax.experimental.pallas import tpu as pltpu
from jax.experimental.pallas import tpu_sc as plsc

_D = 32
_NC = 2
_NS = 16
_NW = _NC * _NS
_S = 2
_NBUF = 4
_HP = 56
_DP = 128


_GC = 40
_TCH = 98


@functools.lru_cache(maxsize=None)
def _build_compact(vocab: int):
    rows_per_chunk = _GC * 8
    n_chunk_total = vocab // rows_per_chunk
    assert vocab % rows_per_chunk == 0 and _TCH * _NW >= n_chunk_total
    mesh = plsc.VectorSubcoreMesh(core_axis_name="c", subcore_axis_name="s")

    @functools.partial(
        pl.kernel,
        mesh=mesh,
        out_type=jax.ShapeDtypeStruct((vocab * _D,), jnp.float32),
        scratch_types=[
            pltpu.VMEM((2, _GC, 8, _D), jnp.float32),
            pltpu.VMEM((2, rows_per_chunk * _D), jnp.float32),
            [pltpu.SemaphoreType.DMA] * 2,
            [pltpu.SemaphoreType.DMA] * 2,
        ],
    )
    def compact(tab_hbm, flat_hbm, vbuf, cbuf, isem, osem):
        wid = lax.axis_index("s") * _NC + lax.axis_index("c")

        def chunk_g0(j):
            return jnp.minimum((wid * _TCH + j) * _GC, vocab // 8 - _GC)

        def i_copy(j, b):
            return pltpu.make_async_copy(
                tab_hbm.at[pl.ds(chunk_g0(j), _GC), :, :], vbuf.at[b],
                isem[b])

        def o_copy(j, b):
            return pltpu.make_async_copy(
                cbuf.at[b],
                flat_hbm.at[pl.ds(chunk_g0(j) * 8 * _D, rows_per_chunk * _D)],
                osem[b])

        def extract(b):
            for g in range(_GC):
                for r in range(8):
                    n = (g * 8 + r) * _D
                    for h in range(0, _D, 16):
                        cbuf[b, pl.ds(n + h, 16)] = vbuf[b, g, r, pl.ds(h, 16)]

        i_copy(0, 0).start()
        i_copy(1, 1).start()

        def pair(j, launch_next):
            for b in range(2):
                jj = 2 * j + b
                i_copy(jj, b).wait()
                extract(b)
                o_copy(jj, b).start()
                o_copy(jj, b).wait()
                if launch_next:
                    i_copy(jj + 2, b).start()

        def body(j, carry):
            pair(j, True)
            return carry

        lax.fori_loop(0, _TCH // 2 - 1, body, 0)
        pair(_TCH // 2 - 1, False)

    return compact


@functools.lru_cache(maxsize=None)
def _build(batch: int, hist: int):
    s_per_w = batch // _NW
    n_chunk = s_per_w // _S
    n_grp = n_chunk // _NBUF
    assert n_chunk % _NBUF == 0 and n_grp >= 3
    mesh = plsc.VectorSubcoreMesh(core_axis_name="c", subcore_axis_name="s")

    @functools.partial(
        pl.kernel,
        mesh=mesh,
        out_type=jax.ShapeDtypeStruct((batch, _HP, _DP), jnp.float32),
        scratch_types=[
            pltpu.VMEM((s_per_w, hist), jnp.int32),
            pltpu.VMEM((_NBUF, _S, hist, _D), jnp.float32),
            pltpu.VMEM((_NBUF, _S, _HP, _DP), jnp.float32),
            [pltpu.SemaphoreType.DMA] * _NBUF,
            [pltpu.SemaphoreType.DMA] * _NBUF,
        ],
        compiler_params=pltpu.CompilerParams(use_tc_tiling_on_sc=False),
    )
    def gather(x_hbm, table_hbm, out_hbm, idx_v, gbuf, pbuf, gsem, wsem):
        wid = lax.axis_index("s") * _NC + lax.axis_index("c")
        base = wid * s_per_w

        def g_copy_one(i, b, k):
            return pltpu.make_async_copy(
                table_hbm.at[idx_v.at[i * _S + k, :]],
                gbuf.at[b].at[k], gsem[b])

        def g_start(i, b):
            for k in range(_S):
                g_copy_one(i, b, k).start()

        def g_wait(i, b):
            for k in range(_S):
                g_copy_one(i, b, k).wait()

        def repack(b):
            for k in range(_S):
                for j in range(hist):
                    for h in range(0, _D, 16):
                        pbuf[b, k, j, pl.ds(h, 16)] = (
                            gbuf[b, k, j, pl.ds(h, 16)])

        def w_copy(i, b):
            return pltpu.make_async_copy(
                pbuf.at[b], out_hbm.at[pl.ds(base + i * _S, _S), :, :],
                wsem[b])

        pltpu.sync_copy(x_hbm.at[pl.ds(base, s_per_w), :], idx_v)

        for b in range(_NBUF):
            g_start(b, b)

        def slots(i0, first, launch_next):
            for b in range(_NBUF):
                i = i0 + b
                g_wait(i, b)
                if not first:
                    w_copy(i - _NBUF, b).wait()
                repack(b)
                w_copy(i, b).start()
                if launch_next:
                    g_start(i + _NBUF, b)

        slots(0, True, True)

        def body(j, carry):
            slots(j * _NBUF, False, True)
            return carry

        lax.fori_loop(1, n_grp - 1, body, 0)
        slots((n_grp - 1) * _NBUF, False, False)
        for b in range(_NBUF):
            w_copy((n_grp - 1) * _NBUF + b, b).wait()

    return gather


def kernel(X, table):
    B, H = X.shape
    V = table.shape[0]
    tab_flat = _build_compact(V)(table.reshape(V // 8, 8, _D))
    out_pad = _build(B, H)(X.astype(jnp.int32), tab_flat.reshape(V, _D))
    return out_pad[:, :H, :_D]
